# baseline (speedup 1.0000x reference)
import functools

import jax
import jax.numpy as jnp
import numpy as np
from jax.experimental import pallas as pl
from jax.experimental.pallas import tpu as pltpu
from jax.experimental.shard_map import shard_map
from jax.sharding import Mesh, PartitionSpec as P_

_SIGMA = 1.0
_B = 1.0
_EPS = 1e-5
_BI = 512


def _cx_kernel(tfull_ref, tloc_ref, i_ref, o_ref, mt_ref, tn_ref, kmax_ref,
               *, nb, p):
    n = pl.program_id(0)
    ib = pl.program_id(1)

    @pl.when(ib == 0)
    def _prologue():
        t_all = tfull_ref[...]
        tot = jnp.sum(jnp.sum(t_all, axis=0), axis=1, keepdims=True)
        mt = tot / (t_all.shape[0] * p)
        mt_ref[...] = mt
        tc = tloc_ref[n] - mt
        tnorm = jnp.sqrt(jnp.sum(tc * tc, axis=0, keepdims=True))
        tn_ref[...] = tc / tnorm
        kmax_ref[...] = jnp.zeros_like(kmax_ref)

    ic = i_ref[0] - mt_ref[...]
    inorm = jnp.sqrt(jnp.sum(ic * ic, axis=0, keepdims=True))
    iu = ic / inorm
    cos = jax.lax.dot_general(
        iu, tn_ref[...],
        dimension_numbers=(((0,), (0,)), ((), ())),
        preferred_element_type=jnp.float32,
    )
    maxcos = jnp.max(cos, axis=1, keepdims=True)
    c2 = 1.0 / (1.0 - maxcos + 2.0 * _EPS)
    c1 = _B - c2
    w = jnp.exp(c1 + c2 * cos)
    s = jnp.sum(w, axis=1, keepdims=True)
    kmax_ref[...] = jnp.maximum(
        kmax_ref[...], jnp.max(w * (1.0 / s), axis=0, keepdims=True))

    @pl.when(ib == nb - 1)
    def _epilogue():
        cs_mean = jnp.sum(kmax_ref[...]) / p
        o_ref[...] = jnp.full(o_ref.shape, -jnp.log(cs_mean), jnp.float32)


def _run(t_full, t_loc, i_loc, *, nb, p):
    n_all, c, _ = t_full.shape
    n_loc = i_loc.shape[0]
    return pl.pallas_call(
        functools.partial(_cx_kernel, nb=nb, p=p),
        grid=(n_loc, nb),
        in_specs=[
            pl.BlockSpec((n_all, c, p), lambda ni, bi: (0, 0, 0)),
            pl.BlockSpec((n_loc, c, p), lambda ni, bi: (0, 0, 0)),
            pl.BlockSpec((1, c, _BI), lambda ni, bi: (ni, 0, bi)),
        ],
        out_specs=pl.BlockSpec((1, 1, 128), lambda ni, bi: (ni, 0, 0)),
        out_shape=jax.ShapeDtypeStruct((n_loc, 1, 128), jnp.float32),
        scratch_shapes=[
            pltpu.VMEM((c, 1), jnp.float32),
            pltpu.VMEM((c, p), jnp.float32),
            pltpu.VMEM((1, p), jnp.float32),
        ],
        compiler_params=pltpu.CompilerParams(
            dimension_semantics=("parallel", "arbitrary"),
            vmem_limit_bytes=56 * 1024 * 1024,
        ),
        name="contextual_loss",
    )(t_full, t_loc, i_loc)


def kernel(I_features, T_features):
    n, c, h, w = I_features.shape
    p = h * w
    i3 = I_features.reshape(n, c, p)
    t3 = T_features.reshape(n, c, p)
    nb = p // _BI

    devs = jax.devices()
    nshard = max(d for d in (1, 2, 4) if d <= len(devs) and n % d == 0)
    run = functools.partial(_run, nb=nb, p=p)
    if nshard > 1:
        mesh = Mesh(np.array(devs[:nshard]), ("x",))
        out = shard_map(
            run, mesh=mesh,
            in_specs=(P_(), P_("x"), P_("x")),
            out_specs=P_("x"),
            check_rep=False,
        )(t3, t3, i3)
    else:
        out = run(t3, t3, i3)
    return jnp.mean(out[:, 0, 0])


# speedup vs baseline: 3.5493x; 3.5493x over previous
import functools

import jax
import jax.numpy as jnp
from jax.experimental import pallas as pl
from jax.experimental.pallas import tpu as pltpu

_SIGMA = 1.0
_B = 1.0
_EPS = 1e-5
_BI = 512
_NEG = -1e30


def _cx_kernel(t_ref, i_ref, o_ref, mt_ref, tn_ref, kmax_ref, *, nb, p):
    n = pl.program_id(0)
    ib = pl.program_id(1)

    @pl.when(ib == 0)
    def _prologue():
        t_all = t_ref[...]
        tot = jnp.sum(jnp.sum(t_all, axis=0), axis=1, keepdims=True)
        mt = tot / (t_all.shape[0] * p)
        mt_ref[...] = mt
        tc = t_ref[n] - mt
        tnorm = jnp.sqrt(jnp.sum(tc * tc, axis=0, keepdims=True))
        tn_ref[...] = tc / tnorm
        kmax_ref[...] = jnp.full(kmax_ref.shape, _NEG, jnp.float32)

    ic = i_ref[0] - mt_ref[...]
    inorm = jnp.sqrt(jnp.sum(ic * ic, axis=0, keepdims=True))
    iu = ic / inorm
    cos = jax.lax.dot_general(
        iu, tn_ref[...],
        dimension_numbers=(((0,), (0,)), ((), ())),
        preferred_element_type=jnp.float32,
    )
    maxcos = jnp.max(cos, axis=1, keepdims=True)
    c2 = 1.0 / (1.0 - maxcos + 2.0 * _EPS)
    c1 = _B - c2
    s = jnp.sum(jnp.exp(c1 + c2 * cos), axis=1, keepdims=True)
    z = (c1 - jnp.log(s)) + c2 * cos
    kmax_ref[...] = jnp.maximum(kmax_ref[...], jnp.max(z, axis=0, keepdims=True))

    @pl.when(ib == nb - 1)
    def _epilogue():
        cs_mean = jnp.sum(jnp.exp(kmax_ref[...])) / p
        o_ref[...] = jnp.full(o_ref.shape, -jnp.log(cs_mean), jnp.float32)


def kernel(I_features, T_features):
    n, c, h, w = I_features.shape
    p = h * w
    i3 = I_features.reshape(n, c, p)
    t3 = T_features.reshape(n, c, p)
    nb = p // _BI

    out = pl.pallas_call(
        functools.partial(_cx_kernel, nb=nb, p=p),
        grid=(n, nb),
        in_specs=[
            pl.BlockSpec((n, c, p), lambda ni, bi: (0, 0, 0)),
            pl.BlockSpec((1, c, _BI), lambda ni, bi: (ni, 0, bi)),
        ],
        out_specs=pl.BlockSpec((1, 1, 128), lambda ni, bi: (ni, 0, 0)),
        out_shape=jax.ShapeDtypeStruct((n, 1, 128), jnp.float32),
        scratch_shapes=[
            pltpu.VMEM((c, 1), jnp.float32),
            pltpu.VMEM((c, p), jnp.float32),
            pltpu.VMEM((1, p), jnp.float32),
        ],
        compiler_params=pltpu.CompilerParams(
            dimension_semantics=("parallel", "arbitrary"),
            vmem_limit_bytes=56 * 1024 * 1024,
        ),
        name="contextual_loss",
    )(t3, i3)
    return jnp.mean(out[:, 0, 0])


# speedup vs baseline: 4.0869x; 1.1515x over previous
import functools

import jax
import jax.numpy as jnp
from jax.experimental import pallas as pl
from jax.experimental.pallas import tpu as pltpu

_SIGMA = 1.0
_B = 1.0
_EPS = 1e-5
_BI = 1024
_BH = 1024
_NEG = -1e30
_LOG2E = 1.4426950408889634


def _cx_kernel(t_ref, i_ref, o_ref, mt_ref, tn_ref, kmax_ref, *, nb, p):
    n = pl.program_id(0)
    ib = pl.program_id(1)

    @pl.when(ib == 0)
    def _prologue():
        t_all = t_ref[...]
        tot = jnp.sum(jnp.sum(t_all, axis=0), axis=1, keepdims=True)
        mt = tot / (t_all.shape[0] * p)
        mt_ref[...] = mt
        tc = t_ref[n] - mt
        tnorm = jnp.sqrt(jnp.sum(tc * tc, axis=0, keepdims=True))
        tn_ref[...] = tc / tnorm
        kmax_ref[...] = jnp.zeros_like(kmax_ref)

    ic = i_ref[0] - mt_ref[...]
    inorm = jnp.sqrt(jnp.sum(ic * ic, axis=0, keepdims=True))
    iu = ic / inorm
    tn = tn_ref[...]
    cos = jax.lax.dot_general(
        iu, tn,
        dimension_numbers=(((0,), (0,)), ((), ())),
        preferred_element_type=jnp.float32,
    )
    maxcos = jnp.max(cos, axis=1, keepdims=True)
    c2 = _LOG2E / (1.0 - maxcos + 2.0 * _EPS)
    c1 = _LOG2E - c2
    e = jnp.exp2(c1 + c2 * cos)
    s = jnp.sum(e, axis=1, keepdims=True)
    kmax_ref[...] = jnp.maximum(
        kmax_ref[...], jnp.max(e * (1.0 / s), axis=0, keepdims=True))

    @pl.when(ib == nb - 1)
    def _epilogue():
        cs_mean = jnp.sum(kmax_ref[...]) / p
        o_ref[...] = jnp.full(o_ref.shape, -jnp.log(cs_mean), jnp.float32)


def kernel(I_features, T_features):
    n, c, h, w = I_features.shape
    p = h * w
    i3 = I_features.reshape(n, c, p)
    t3 = T_features.reshape(n, c, p)
    nb = p // _BI

    out = pl.pallas_call(
        functools.partial(_cx_kernel, nb=nb, p=p),
        grid=(n, nb),
        in_specs=[
            pl.BlockSpec((n, c, p), lambda ni, bi: (0, 0, 0)),
            pl.BlockSpec((1, c, _BI), lambda ni, bi: (ni, 0, bi)),
        ],
        out_specs=pl.BlockSpec((1, 1, 128), lambda ni, bi: (ni, 0, 0)),
        out_shape=jax.ShapeDtypeStruct((n, 1, 128), jnp.float32),
        scratch_shapes=[
            pltpu.VMEM((c, 1), jnp.float32),
            pltpu.VMEM((c, p), jnp.float32),
            pltpu.VMEM((1, p), jnp.float32),
        ],
        compiler_params=pltpu.CompilerParams(
            dimension_semantics=("parallel", "arbitrary"),
            vmem_limit_bytes=56 * 1024 * 1024,
        ),
        name="contextual_loss",
    )(t3, i3)
    return jnp.mean(out[:, 0, 0])


# speedup vs baseline: 4.1481x; 1.0150x over previous
import functools

import jax
import jax.numpy as jnp
from jax.experimental import pallas as pl
from jax.experimental.pallas import tpu as pltpu

_SIGMA = 1.0
_B = 1.0
_EPS = 1e-5
_BI = 1024
_BH = 256
_NEG = -1e30
_LOG2E = 1.4426950408889634


def _cx_kernel(t_ref, i_ref, o_ref, mt_ref, tn_ref, kmax_ref, *, nb, p):
    n = pl.program_id(0)
    ib = pl.program_id(1)

    @pl.when(ib == 0)
    def _prologue():
        t_all = t_ref[...]
        tot = jnp.sum(jnp.sum(t_all, axis=0), axis=1, keepdims=True)
        mt = tot / (t_all.shape[0] * p)
        mt_ref[...] = mt
        tc = t_ref[n] - mt
        tnorm = jnp.sqrt(jnp.sum(tc * tc, axis=0, keepdims=True))
        tn_ref[...] = tc / tnorm
        kmax_ref[...] = jnp.zeros_like(kmax_ref)

    ic = i_ref[0] - mt_ref[...]
    inorm = jnp.sqrt(jnp.sum(ic * ic, axis=0, keepdims=True))
    iu = ic / inorm
    tn = tn_ref[...]
    acc = kmax_ref[...]
    for j in range(_BI // _BH):
        cos = jax.lax.dot_general(
            iu[:, j * _BH:(j + 1) * _BH], tn,
            dimension_numbers=(((0,), (0,)), ((), ())),
            preferred_element_type=jnp.float32,
        )
        maxcos = jnp.max(cos, axis=1, keepdims=True)
        c2 = _LOG2E / (1.0 - maxcos + 2.0 * _EPS)
        c1 = _LOG2E - c2
        e = jnp.exp2(c1 + c2 * cos)
        s = jnp.sum(e, axis=1, keepdims=True)
        acc = jnp.maximum(acc, jnp.max(e * (1.0 / s), axis=0, keepdims=True))
    kmax_ref[...] = acc

    @pl.when(ib == nb - 1)
    def _epilogue():
        cs_mean = jnp.sum(kmax_ref[...]) / p
        o_ref[...] = jnp.full(o_ref.shape, -jnp.log(cs_mean), jnp.float32)


def kernel(I_features, T_features):
    n, c, h, w = I_features.shape
    p = h * w
    i3 = I_features.reshape(n, c, p)
    t3 = T_features.reshape(n, c, p)
    nb = p // _BI

    out = pl.pallas_call(
        functools.partial(_cx_kernel, nb=nb, p=p),
        grid=(n, nb),
        in_specs=[
            pl.BlockSpec((n, c, p), lambda ni, bi: (0, 0, 0)),
            pl.BlockSpec((1, c, _BI), lambda ni, bi: (ni, 0, bi)),
        ],
        out_specs=pl.BlockSpec((1, 1, 128), lambda ni, bi: (ni, 0, 0)),
        out_shape=jax.ShapeDtypeStruct((n, 1, 128), jnp.float32),
        scratch_shapes=[
            pltpu.VMEM((c, 1), jnp.float32),
            pltpu.VMEM((c, p), jnp.float32),
            pltpu.VMEM((1, p), jnp.float32),
        ],
        compiler_params=pltpu.CompilerParams(
            dimension_semantics=("parallel", "arbitrary"),
            vmem_limit_bytes=56 * 1024 * 1024,
        ),
        name="contextual_loss",
    )(t3, i3)
    return jnp.mean(out[:, 0, 0])


# speedup vs baseline: 4.1603x; 1.0030x over previous
import functools

import jax
import jax.numpy as jnp
from jax.experimental import pallas as pl
from jax.experimental.pallas import tpu as pltpu

_SIGMA = 1.0
_B = 1.0
_EPS = 1e-5
_BI = 1024
_BH = 256
_NEG = -1e30
_LOG2E = 1.4426950408889634


def _cx_kernel(t_ref, i_ref, o_ref, mt_ref, tn_ref, kmax_ref, *, nb, p):
    n = pl.program_id(0)
    ib = pl.program_id(1)

    @pl.when(ib == 0)
    def _prologue():
        t_all = t_ref[...]
        tot = jnp.sum(jnp.sum(t_all, axis=0), axis=1, keepdims=True)
        mt = tot / (t_all.shape[0] * p)
        mt_ref[...] = mt
        tc = t_ref[n] - mt
        tnorm = jnp.sqrt(jnp.sum(tc * tc, axis=0, keepdims=True))
        tn_ref[...] = tc / tnorm
        kmax_ref[...] = jnp.zeros_like(kmax_ref)

    ic = i_ref[0] - mt_ref[...]
    inorm = jnp.sqrt(jnp.sum(ic * ic, axis=0, keepdims=True))
    iu = ic / inorm
    tn = tn_ref[...]
    acc = kmax_ref[...]
    for j in range(_BI // _BH):
        cos = jax.lax.dot_general(
            iu[:, j * _BH:(j + 1) * _BH], tn,
            dimension_numbers=(((0,), (0,)), ((), ())),
            preferred_element_type=jnp.float32,
        )
        maxcos = jnp.max(cos, axis=1, keepdims=True)
        c2 = _LOG2E / (1.0 - maxcos + 2.0 * _EPS)
        c1 = _LOG2E - c2
        e = jnp.exp2(c1 + c2 * cos)
        s = jnp.sum(e, axis=1, keepdims=True)
        cs = (e * (1.0 / s)).reshape(_BH // 8, 8, e.shape[1])
        acc = jnp.maximum(acc, jnp.max(cs, axis=0))
    kmax_ref[...] = acc

    @pl.when(ib == nb - 1)
    def _epilogue():
        cs_mean = jnp.sum(jnp.max(kmax_ref[...], axis=0)) / p
        o_ref[...] = jnp.full(o_ref.shape, -jnp.log(cs_mean), jnp.float32)


def kernel(I_features, T_features):
    n, c, h, w = I_features.shape
    p = h * w
    i3 = I_features.reshape(n, c, p)
    t3 = T_features.reshape(n, c, p)
    nb = p // _BI

    out = pl.pallas_call(
        functools.partial(_cx_kernel, nb=nb, p=p),
        grid=(n, nb),
        in_specs=[
            pl.BlockSpec((n, c, p), lambda ni, bi: (0, 0, 0)),
            pl.BlockSpec((1, c, _BI), lambda ni, bi: (ni, 0, bi)),
        ],
        out_specs=pl.BlockSpec((1, 1, 128), lambda ni, bi: (ni, 0, 0)),
        out_shape=jax.ShapeDtypeStruct((n, 1, 128), jnp.float32),
        scratch_shapes=[
            pltpu.VMEM((c, 1), jnp.float32),
            pltpu.VMEM((c, p), jnp.float32),
            pltpu.VMEM((8, p), jnp.float32),
        ],
        compiler_params=pltpu.CompilerParams(
            dimension_semantics=("parallel", "arbitrary"),
            vmem_limit_bytes=56 * 1024 * 1024,
        ),
        name="contextual_loss",
    )(t3, i3)
    return jnp.mean(out[:, 0, 0])


# speedup vs baseline: 4.1886x; 1.0068x over previous
import functools

import jax
import jax.numpy as jnp
from jax.experimental import pallas as pl
from jax.experimental.pallas import tpu as pltpu

_SIGMA = 1.0
_B = 1.0
_EPS = 1e-5
_BI = 1024
_BH = 256
_NEG = -1e30
_LOG2E = 1.4426950408889634


def _cx_kernel(t_ref, i_ref, o_ref, tn_ref, iu_ref, kmax_ref, *, nb, p):
    n = pl.program_id(0)
    ib = pl.program_id(1)

    @pl.when(ib == 0)
    def _prologue():
        t_all = t_ref[...]
        tot = jnp.sum(jnp.sum(t_all, axis=0), axis=1, keepdims=True)
        mt = tot / (t_all.shape[0] * p)
        tc = t_ref[n] - mt
        tnorm = jnp.sqrt(jnp.sum(tc * tc, axis=0, keepdims=True))
        tn_ref[...] = tc / tnorm
        ic = i_ref[0] - mt
        inorm = jnp.sqrt(jnp.sum(ic * ic, axis=0, keepdims=True))
        iu_ref[...] = ic / inorm
        kmax_ref[...] = jnp.zeros_like(kmax_ref)

    tn = tn_ref[...]
    acc = kmax_ref[...]
    for j in range(_BI // _BH):
        cos = jax.lax.dot_general(
            iu_ref[:, pl.ds(ib * _BI + j * _BH, _BH)], tn,
            dimension_numbers=(((0,), (0,)), ((), ())),
            preferred_element_type=jnp.float32,
        )
        maxcos = jnp.max(cos, axis=1, keepdims=True)
        c2 = _LOG2E / (1.0 - maxcos + 2.0 * _EPS)
        c1 = _LOG2E - c2
        e = jnp.exp2(c1 + c2 * cos)
        s = jnp.sum(e, axis=1, keepdims=True)
        cs = (e * (1.0 / s)).reshape(_BH // 8, 8, e.shape[1])
        acc = jnp.maximum(acc, jnp.max(cs, axis=0))
    kmax_ref[...] = acc

    @pl.when(ib == nb - 1)
    def _epilogue():
        cs_mean = jnp.sum(jnp.max(kmax_ref[...], axis=0)) / p
        o_ref[...] = jnp.full(o_ref.shape, -jnp.log(cs_mean), jnp.float32)


def kernel(I_features, T_features):
    n, c, h, w = I_features.shape
    p = h * w
    i3 = I_features.reshape(n, c, p)
    t3 = T_features.reshape(n, c, p)
    nb = p // _BI

    out = pl.pallas_call(
        functools.partial(_cx_kernel, nb=nb, p=p),
        grid=(n, nb),
        in_specs=[
            pl.BlockSpec((n, c, p), lambda ni, bi: (0, 0, 0)),
            pl.BlockSpec((1, c, p), lambda ni, bi: (ni, 0, 0)),
        ],
        out_specs=pl.BlockSpec((1, 1, 128), lambda ni, bi: (ni, 0, 0)),
        out_shape=jax.ShapeDtypeStruct((n, 1, 128), jnp.float32),
        scratch_shapes=[
            pltpu.VMEM((c, p), jnp.float32),
            pltpu.VMEM((c, p), jnp.float32),
            pltpu.VMEM((8, p), jnp.float32),
        ],
        compiler_params=pltpu.CompilerParams(
            dimension_semantics=("parallel", "arbitrary"),
            vmem_limit_bytes=56 * 1024 * 1024,
        ),
        name="contextual_loss",
    )(t3, i3)
    return jnp.mean(out[:, 0, 0])
